# Initial kernel scaffold; baseline (speedup 1.0000x reference)
import jax
import jax.numpy as jnp
from jax.experimental import pallas as pl
from jax.experimental.pallas import tpu as pltpu

_OUT_SIZE = 9


def _combine_body(mu_ref, m_ref, M_ref, cg_ref, nu_ref, b1_ref, sel0_ref,
                  sel1_ref, out_ref, c_scr):
    lam_dim, s_blk, nf = nu_ref.shape
    l_dim = b1_ref.shape[0]
    q = sel0_ref.shape[1]
    i = pl.program_id(0)
    n_terms = mu_ref.shape[0]

    @pl.when(i == 0)
    def _():
        def zero_body(k, carry):
            c_scr[k] = 0.0
            return carry
        jax.lax.fori_loop(0, lam_dim * l_dim * _OUT_SIZE, zero_body, 0)

        def scat_body(t, carry):
            idx = mu_ref[t] * (l_dim * _OUT_SIZE) + m_ref[t] * _OUT_SIZE + M_ref[t]
            c_scr[idx] = c_scr[idx] + cg_ref[t]
            return carry
        jax.lax.fori_loop(0, n_terms, scat_body, 0)

    iota_f = jax.lax.broadcasted_iota(jnp.int32, (nf, q), 0)
    g0 = (iota_f == sel0_ref[...]).astype(jnp.bfloat16)
    g1 = (iota_f == sel1_ref[...]).astype(jnp.bfloat16)

    nu_bf = nu_ref[...].astype(jnp.bfloat16).reshape(lam_dim * s_blk, nf)
    b1_bf = b1_ref[...].astype(jnp.bfloat16).reshape(l_dim * s_blk, nf)
    asel = jnp.dot(nu_bf, g0, preferred_element_type=jnp.float32)
    bsel = jnp.dot(b1_bf, g1, preferred_element_type=jnp.float32)

    accs = [None] * _OUT_SIZE
    for x in range(lam_dim):
        ax = asel[x * s_blk:(x + 1) * s_blk, :]
        for y in range(l_dim):
            p = ax * bsel[y * s_blk:(y + 1) * s_blk, :]
            base = (x * l_dim + y) * _OUT_SIZE
            for mm in range(_OUT_SIZE):
                t = p * c_scr[base + mm]
                accs[mm] = t if accs[mm] is None else accs[mm] + t
    for mm in range(_OUT_SIZE):
        out_ref[mm] = accs[mm]


def kernel(block_nu_values, block_1_values, selected_features, mu_array,
           m_array, M_array, cg_array):
    n, lam_dim, _ = block_nu_values.shape
    l_dim = block_1_values.shape[1]
    nf = block_1_values.shape[2]
    q = selected_features.shape[0]

    s_blk = 160
    assert n % s_blk == 0
    grid = (n // s_blk,)

    nu_t = block_nu_values[:, :, :nf].transpose(1, 0, 2)
    b1_t = block_1_values.transpose(1, 0, 2)
    sel0 = selected_features[:, 0].reshape(1, q).astype(jnp.int32)
    sel1 = selected_features[:, 1].reshape(1, q).astype(jnp.int32)

    grid_spec = pltpu.PrefetchScalarGridSpec(
        num_scalar_prefetch=4,
        grid=grid,
        in_specs=[
            pl.BlockSpec((lam_dim, s_blk, nf), lambda i, *_: (0, i, 0)),
            pl.BlockSpec((l_dim, s_blk, nf), lambda i, *_: (0, i, 0)),
            pl.BlockSpec((1, q), lambda i, *_: (0, 0)),
            pl.BlockSpec((1, q), lambda i, *_: (0, 0)),
        ],
        out_specs=pl.BlockSpec((_OUT_SIZE, s_blk, q), lambda i, *_: (0, i, 0)),
        scratch_shapes=[pltpu.SMEM((lam_dim * l_dim * _OUT_SIZE,), jnp.float32)],
    )
    out_t = pl.pallas_call(
        _combine_body,
        grid_spec=grid_spec,
        out_shape=jax.ShapeDtypeStruct((_OUT_SIZE, n, q), jnp.float32),
        compiler_params=pltpu.CompilerParams(
            dimension_semantics=("arbitrary",),
        ),
    )(mu_array, m_array, M_array, cg_array, nu_t, b1_t, sel0, sel1)
    return out_t.transpose(1, 0, 2)


# speedup vs baseline: 1.7799x; 1.7799x over previous
import jax
import jax.numpy as jnp
from jax.experimental import pallas as pl
from jax.experimental.pallas import tpu as pltpu

_OUT_SIZE = 9
_SUB_S = 8
_SUB_Q = 256


def _combine_body(mu_ref, m_ref, M_ref, cg_ref, nu_ref, b1_ref, sel0_ref,
                  sel1_ref, out_ref, asel_scr, bsel_scr, c_scr):
    lam_dim, s_blk, nf = nu_ref.shape
    l_dim = b1_ref.shape[0]
    q = sel0_ref.shape[1]
    i = pl.program_id(0)
    n_terms = mu_ref.shape[0]

    @pl.when(i == 0)
    def _():
        def zero_body(k, carry):
            c_scr[k] = 0.0
            return carry
        jax.lax.fori_loop(0, lam_dim * l_dim * _OUT_SIZE, zero_body, 0)

        def scat_body(t, carry):
            idx = mu_ref[t] * (l_dim * _OUT_SIZE) + m_ref[t] * _OUT_SIZE + M_ref[t]
            c_scr[idx] = c_scr[idx] + cg_ref[t]
            return carry
        jax.lax.fori_loop(0, n_terms, scat_body, 0)

    iota_f = jax.lax.broadcasted_iota(jnp.int32, (nf, q), 0)
    g0 = (iota_f == sel0_ref[...]).astype(jnp.bfloat16)
    g1 = (iota_f == sel1_ref[...]).astype(jnp.bfloat16)
    for x in range(lam_dim):
        asel_scr[x * s_blk:(x + 1) * s_blk, :] = jnp.dot(
            nu_ref[x].astype(jnp.bfloat16), g0,
            preferred_element_type=jnp.float32)
    for y in range(l_dim):
        bsel_scr[y * s_blk:(y + 1) * s_blk, :] = jnp.dot(
            b1_ref[y].astype(jnp.bfloat16), g1,
            preferred_element_type=jnp.float32)

    def sub_body(j, carry):
        row = j * _SUB_S
        for qh in range(q // _SUB_Q):
            qs = qh * _SUB_Q
            bys = [bsel_scr[pl.ds(y * s_blk + row, _SUB_S), qs:qs + _SUB_Q]
                   for y in range(l_dim)]
            accs = [None] * _OUT_SIZE
            for x in range(lam_dim):
                ax = asel_scr[pl.ds(x * s_blk + row, _SUB_S), qs:qs + _SUB_Q]
                for y in range(l_dim):
                    p = ax * bys[y]
                    base = (x * l_dim + y) * _OUT_SIZE
                    for mm in range(_OUT_SIZE):
                        t = p * c_scr[base + mm]
                        accs[mm] = t if accs[mm] is None else accs[mm] + t
            for mm in range(_OUT_SIZE):
                out_ref[mm, pl.ds(row, _SUB_S), qs:qs + _SUB_Q] = accs[mm]
        return carry

    jax.lax.fori_loop(0, s_blk // _SUB_S, sub_body, 0)


def kernel(block_nu_values, block_1_values, selected_features, mu_array,
           m_array, M_array, cg_array):
    n, lam_dim, _ = block_nu_values.shape
    l_dim = block_1_values.shape[1]
    nf = block_1_values.shape[2]
    q = selected_features.shape[0]

    s_blk = 160
    assert n % s_blk == 0
    grid = (n // s_blk,)

    nu_t = block_nu_values[:, :, :nf].transpose(1, 0, 2)
    b1_t = block_1_values.transpose(1, 0, 2)
    sel0 = selected_features[:, 0].reshape(1, q).astype(jnp.int32)
    sel1 = selected_features[:, 1].reshape(1, q).astype(jnp.int32)

    grid_spec = pltpu.PrefetchScalarGridSpec(
        num_scalar_prefetch=4,
        grid=grid,
        in_specs=[
            pl.BlockSpec((lam_dim, s_blk, nf), lambda i, *_: (0, i, 0)),
            pl.BlockSpec((l_dim, s_blk, nf), lambda i, *_: (0, i, 0)),
            pl.BlockSpec((1, q), lambda i, *_: (0, 0)),
            pl.BlockSpec((1, q), lambda i, *_: (0, 0)),
        ],
        out_specs=pl.BlockSpec((_OUT_SIZE, s_blk, q), lambda i, *_: (0, i, 0)),
        scratch_shapes=[
            pltpu.VMEM((lam_dim * s_blk, q), jnp.float32),
            pltpu.VMEM((l_dim * s_blk, q), jnp.float32),
            pltpu.SMEM((lam_dim * l_dim * _OUT_SIZE,), jnp.float32),
        ],
    )
    out_t = pl.pallas_call(
        _combine_body,
        grid_spec=grid_spec,
        out_shape=jax.ShapeDtypeStruct((_OUT_SIZE, n, q), jnp.float32),
        compiler_params=pltpu.CompilerParams(
            dimension_semantics=("arbitrary",),
        ),
    )(mu_array, m_array, M_array, cg_array, nu_t, b1_t, sel0, sel1)
    return out_t.transpose(1, 0, 2)


# speedup vs baseline: 2.2798x; 1.2808x over previous
import jax
import jax.numpy as jnp
from jax.experimental import pallas as pl
from jax.experimental.pallas import tpu as pltpu

_OUT_SIZE = 9
_SUB_S = 8
_SUB_Q = 256


def _combine_body(mu_ref, m_ref, M_ref, cg_ref, nu_ref, b1_ref, sel0_ref,
                  sel1_ref, out_ref, asel_scr, bsel_scr, c_scr):
    lam_dim, s_blk, nf = nu_ref.shape
    l_dim = b1_ref.shape[0]
    q = sel0_ref.shape[1]
    i = pl.program_id(0)
    n_terms = mu_ref.shape[0]

    @pl.when(i == 0)
    def _():
        def zero_body(k, carry):
            c_scr[k] = 0.0
            return carry
        jax.lax.fori_loop(0, lam_dim * l_dim * _OUT_SIZE, zero_body, 0)

        def scat_body(t, carry):
            idx = mu_ref[t] * (l_dim * _OUT_SIZE) + m_ref[t] * _OUT_SIZE + M_ref[t]
            c_scr[idx] = c_scr[idx] + cg_ref[t]
            return carry
        jax.lax.fori_loop(0, n_terms, scat_body, 0)

    iota_f = jax.lax.broadcasted_iota(jnp.int32, (nf, q), 0)
    g0 = (iota_f == sel0_ref[...]).astype(jnp.bfloat16)
    g1 = (iota_f == sel1_ref[...]).astype(jnp.bfloat16)
    for x in range(lam_dim):
        asel_scr[x * s_blk:(x + 1) * s_blk, :] = jnp.dot(
            nu_ref[x].astype(jnp.bfloat16), g0,
            preferred_element_type=jnp.float32)
    for y in range(l_dim):
        bsel_scr[y * s_blk:(y + 1) * s_blk, :] = jnp.dot(
            b1_ref[y].astype(jnp.bfloat16), g1,
            preferred_element_type=jnp.float32)

    def sub_body(j, carry):
        row = j * _SUB_S
        cs = [c_scr[k] for k in range(lam_dim * l_dim * _OUT_SIZE)]
        for qh in range(q // _SUB_Q):
            qs = qh * _SUB_Q
            bys = [bsel_scr[pl.ds(y * s_blk + row, _SUB_S), qs:qs + _SUB_Q]
                   for y in range(l_dim)]
            accs = [None] * _OUT_SIZE
            for x in range(lam_dim):
                ax = asel_scr[pl.ds(x * s_blk + row, _SUB_S), qs:qs + _SUB_Q]
                for y in range(l_dim):
                    p = ax * bys[y]
                    base = (x * l_dim + y) * _OUT_SIZE
                    for mm in range(_OUT_SIZE):
                        t = p * cs[base + mm]
                        accs[mm] = t if accs[mm] is None else accs[mm] + t
            for mm in range(_OUT_SIZE):
                out_ref[mm, pl.ds(row, _SUB_S), qs:qs + _SUB_Q] = accs[mm]
        return carry

    jax.lax.fori_loop(0, s_blk // _SUB_S, sub_body, 0)


def kernel(block_nu_values, block_1_values, selected_features, mu_array,
           m_array, M_array, cg_array):
    n, lam_dim, _ = block_nu_values.shape
    l_dim = block_1_values.shape[1]
    nf = block_1_values.shape[2]
    q = selected_features.shape[0]

    s_blk = 160
    assert n % s_blk == 0
    grid = (n // s_blk,)

    nu_t = block_nu_values[:, :, :nf].transpose(1, 0, 2)
    b1_t = block_1_values.transpose(1, 0, 2)
    sel0 = selected_features[:, 0].reshape(1, q).astype(jnp.int32)
    sel1 = selected_features[:, 1].reshape(1, q).astype(jnp.int32)

    grid_spec = pltpu.PrefetchScalarGridSpec(
        num_scalar_prefetch=4,
        grid=grid,
        in_specs=[
            pl.BlockSpec((lam_dim, s_blk, nf), lambda i, *_: (0, i, 0)),
            pl.BlockSpec((l_dim, s_blk, nf), lambda i, *_: (0, i, 0)),
            pl.BlockSpec((1, q), lambda i, *_: (0, 0)),
            pl.BlockSpec((1, q), lambda i, *_: (0, 0)),
        ],
        out_specs=pl.BlockSpec((_OUT_SIZE, s_blk, q), lambda i, *_: (0, i, 0)),
        scratch_shapes=[
            pltpu.VMEM((lam_dim * s_blk, q), jnp.float32),
            pltpu.VMEM((l_dim * s_blk, q), jnp.float32),
            pltpu.SMEM((lam_dim * l_dim * _OUT_SIZE,), jnp.float32),
        ],
    )
    out_t = pl.pallas_call(
        _combine_body,
        grid_spec=grid_spec,
        out_shape=jax.ShapeDtypeStruct((_OUT_SIZE, n, q), jnp.float32),
        compiler_params=pltpu.CompilerParams(
            dimension_semantics=("arbitrary",),
        ),
    )(mu_array, m_array, M_array, cg_array, nu_t, b1_t, sel0, sel1)
    return out_t.transpose(1, 0, 2)


# speedup vs baseline: 5.2138x; 2.2870x over previous
import jax
import jax.numpy as jnp
from jax.experimental import pallas as pl
from jax.experimental.pallas import tpu as pltpu

_OUT_SIZE = 9
_SUB_S = 8
_SUB_Q = 256


def _combine_body(mu_ref, m_ref, M_ref, cg_ref, nu_ref, b1_ref, sel0_ref,
                  sel1_ref, out_ref, nc_scr, ncsel_scr, bsel_scr, c_scr):
    lam_dim, s_blk, nf = nu_ref.shape
    l_dim = b1_ref.shape[0]
    q = sel0_ref.shape[1]
    i = pl.program_id(0)
    n_terms = mu_ref.shape[0]
    n_ch = l_dim * _OUT_SIZE

    @pl.when(i == 0)
    def _():
        def zero_body(k, carry):
            c_scr[k] = 0.0
            return carry
        jax.lax.fori_loop(0, lam_dim * l_dim * _OUT_SIZE, zero_body, 0)

        def scat_body(t, carry):
            idx = mu_ref[t] * (l_dim * _OUT_SIZE) + m_ref[t] * _OUT_SIZE + M_ref[t]
            c_scr[idx] = c_scr[idx] + cg_ref[t]
            return carry
        jax.lax.fori_loop(0, n_terms, scat_body, 0)

    nus = [nu_ref[x] for x in range(lam_dim)]
    for y in range(l_dim):
        for mm in range(_OUT_SIZE):
            acc = None
            for x in range(lam_dim):
                t = nus[x] * c_scr[(x * l_dim + y) * _OUT_SIZE + mm]
                acc = t if acc is None else acc + t
            ch = y * _OUT_SIZE + mm
            nc_scr[ch * s_blk:(ch + 1) * s_blk, :] = acc.astype(jnp.bfloat16)

    iota_f = jax.lax.broadcasted_iota(jnp.int32, (nf, q), 0)
    g0 = (iota_f == sel0_ref[...]).astype(jnp.bfloat16)
    g1 = (iota_f == sel1_ref[...]).astype(jnp.bfloat16)
    for y in range(l_dim):
        base = y * _OUT_SIZE * s_blk
        ncsel_scr[base:base + _OUT_SIZE * s_blk, :] = jnp.dot(
            nc_scr[base:base + _OUT_SIZE * s_blk, :], g0,
            preferred_element_type=jnp.float32)
    bsel_scr[...] = jnp.dot(
        b1_ref[...].astype(jnp.bfloat16).reshape(l_dim * s_blk, nf), g1,
        preferred_element_type=jnp.float32)

    def sub_body(j, carry):
        row = j * _SUB_S
        for qh in range(q // _SUB_Q):
            qs = qh * _SUB_Q
            bys = [bsel_scr[pl.ds(y * s_blk + row, _SUB_S), qs:qs + _SUB_Q]
                   for y in range(l_dim)]
            for mm in range(_OUT_SIZE):
                acc = None
                for y in range(l_dim):
                    ch = y * _OUT_SIZE + mm
                    nct = ncsel_scr[pl.ds(ch * s_blk + row, _SUB_S),
                                    qs:qs + _SUB_Q]
                    t = nct * bys[y]
                    acc = t if acc is None else acc + t
                out_ref[mm, pl.ds(row, _SUB_S), qs:qs + _SUB_Q] = acc
        return carry

    jax.lax.fori_loop(0, s_blk // _SUB_S, sub_body, 0)


def kernel(block_nu_values, block_1_values, selected_features, mu_array,
           m_array, M_array, cg_array):
    n, lam_dim, _ = block_nu_values.shape
    l_dim = block_1_values.shape[1]
    nf = block_1_values.shape[2]
    q = selected_features.shape[0]

    s_blk = 80
    assert n % s_blk == 0
    grid = (n // s_blk,)

    nu_t = block_nu_values[:, :, :nf].transpose(1, 0, 2)
    b1_t = block_1_values.transpose(1, 0, 2)
    sel0 = selected_features[:, 0].reshape(1, q).astype(jnp.int32)
    sel1 = selected_features[:, 1].reshape(1, q).astype(jnp.int32)

    grid_spec = pltpu.PrefetchScalarGridSpec(
        num_scalar_prefetch=4,
        grid=grid,
        in_specs=[
            pl.BlockSpec((lam_dim, s_blk, nf), lambda i, *_: (0, i, 0)),
            pl.BlockSpec((l_dim, s_blk, nf), lambda i, *_: (0, i, 0)),
            pl.BlockSpec((1, q), lambda i, *_: (0, 0)),
            pl.BlockSpec((1, q), lambda i, *_: (0, 0)),
        ],
        out_specs=pl.BlockSpec((_OUT_SIZE, s_blk, q), lambda i, *_: (0, i, 0)),
        scratch_shapes=[
            pltpu.VMEM((l_dim * _OUT_SIZE * s_blk, nf), jnp.bfloat16),
            pltpu.VMEM((l_dim * _OUT_SIZE * s_blk, q), jnp.float32),
            pltpu.VMEM((l_dim * s_blk, q), jnp.float32),
            pltpu.SMEM((lam_dim * l_dim * _OUT_SIZE,), jnp.float32),
        ],
    )
    out_t = pl.pallas_call(
        _combine_body,
        grid_spec=grid_spec,
        out_shape=jax.ShapeDtypeStruct((_OUT_SIZE, n, q), jnp.float32),
        compiler_params=pltpu.CompilerParams(
            dimension_semantics=("arbitrary",),
        ),
    )(mu_array, m_array, M_array, cg_array, nu_t, b1_t, sel0, sel1)
    return out_t.transpose(1, 0, 2)
